# baseline (speedup 1.0000x reference)
import functools

import jax
import jax.numpy as jnp
from jax import lax
from jax.experimental import pallas as pl
from jax.experimental.pallas import tpu as pltpu
from jax.experimental.pallas import tpu_sc as plsc

F32 = jnp.float32
I32 = jnp.int32

D_OBJ0 = 132
D_HI = 16
D_PRED = 128
D_OUT = 128
HID = 512

NC, NS = 2, 16
NW = NC * NS
EB = 128
NB = 13
E_W = EB * NB
E_PAD = NW * E_W
R_PAD = 50176
ROWS_T = R_PAD // NS
DEAD = 50100
CH = 16
CW = 32
BN = 512

NBP = 16

E4 = E_PAD // 4
R4 = R_PAD // 4


def _sc_mesh():
    return plsc.VectorSubcoreMesh(core_axis_name="c", subcore_axis_name="s",
                                  num_cores=NC, num_subcores=NS)


@functools.cache
def _make_gather(d, dh=None):
    nbuf = 6
    outs = [jax.ShapeDtypeStruct((E_PAD, d), F32)] * 2
    scratch = [
        pltpu.VMEM((1, NBP, EB), I32),
        pltpu.VMEM((1, NBP, EB), I32),
    ] + [pltpu.VMEM((EB, d), F32)] * nbuf + [pltpu.SemaphoreType.DMA] * nbuf
    if dh is not None:
        outs += [jax.ShapeDtypeStruct((E_PAD, dh), F32)] * 2
        scratch += [pltpu.VMEM((EB, dh), F32)] * nbuf \
            + [pltpu.SemaphoreType.DMA] * nbuf

    @functools.partial(
        pl.kernel,
        out_type=tuple(outs),
        mesh=_sc_mesh(),
        compiler_params=pltpu.CompilerParams(use_tc_tiling_on_sc=False),
        scratch_types=scratch,
    )
    def gather(*args):
        if dh is not None:
            table, table_h, sidx, oidx, gs, go, gsh, goh, idx_sv, idx_ov, *bs = args
            bufs_h = bs[2 * nbuf:3 * nbuf]
            sems_h = bs[3 * nbuf:]
        else:
            table, sidx, oidx, gs, go, idx_sv, idx_ov, *bs = args
        bufs = bs[:nbuf]
        sems_in = bs[nbuf:2 * nbuf]
        c = lax.axis_index("c")
        s = lax.axis_index("s")
        wid = c * NS + s
        pltpu.sync_copy(sidx.at[pl.ds(wid, 1)], idx_sv)
        pltpu.sync_copy(oidx.at[pl.ds(wid, 1)], idx_ov)
        base = wid * E_W

        def _pipeline(tbl, steps, rbufs, rsems):
            L = len(steps)

            def _issue(k):
                idxv, _, b = steps[k]
                return pltpu.async_copy(tbl.at[idxv.at[0, b]],
                                        rbufs[k % nbuf], rsems[k % nbuf])

            pend = {k: _issue(k) for k in range(min(nbuf, L))}
            for k in range(L):
                pend.pop(k).wait()
                _, out, b = steps[k]
                pltpu.sync_copy(rbufs[k % nbuf],
                                out.at[pl.ds(base + b * EB, EB)])
                if k + nbuf < L:
                    pend[k + nbuf] = _issue(k + nbuf)

        _pipeline(table,
                  [(idx_sv, gs, b) for b in range(NB)]
                  + [(idx_ov, go, b) for b in range(NB)], bufs, sems_in)
        if dh is not None:
            _pipeline(table_h,
                      [(idx_sv, gsh, b) for b in range(NB)]
                      + [(idx_ov, goh, b) for b in range(NB)], bufs_h, sems_h)

    return gather


@functools.cache
def _make_scatter():
    nsb = 4

    @functools.partial(
        pl.kernel,
        out_type=jax.ShapeDtypeStruct((CH, R_PAD, CW), F32),
        mesh=_sc_mesh(),
        compiler_params=pltpu.CompilerParams(use_tc_tiling_on_sc=False),
        scratch_types=[
            pltpu.VMEM_SHARED((R_PAD, CW), F32),
            pltpu.VMEM((2, NBP, EB), I32),
            pltpu.VMEM((2, NBP, EB), I32),
        ] + [pltpu.VMEM((EB, CW), F32)] * nsb
          + [pltpu.SemaphoreType.DMA] * nsb,
    )
    def scatter(src_s, src_o, sidx, oidx, zrows, pooled,
                acc, idx_sv, idx_ov, *bufsems):
        bufs = bufsems[:nsb]
        sems_st = bufsems[nsb:]
        c = lax.axis_index("c")
        s = lax.axis_index("s")
        pltpu.sync_copy(sidx.at[pl.ds(s * 2, 2)], idx_sv)
        pltpu.sync_copy(oidx.at[pl.ds(s * 2, 2)], idx_ov)
        row0 = s * ROWS_T
        ebase = s * (2 * NB * EB)
        steps = [(src_s, idx_sv, r) for r in range(2 * NB)] + \
                [(src_o, idx_ov, r) for r in range(2 * NB)]
        L = len(steps)

        def chunk_body(jj, carry):
            j = c * (CH // NC) + jj
            pltpu.sync_copy(zrows, acc.at[pl.ds(row0, ROWS_T)])
            plsc.subcore_barrier()

            def _issue(k):
                src, _, r = steps[k]
                return pltpu.async_copy(src.at[j, pl.ds(ebase + r * EB, EB)],
                                        bufs[k % nsb], sems_st[k % nsb])

            pend = {k: _issue(k) for k in range(min(nsb, L))}
            for k in range(L):
                pend.pop(k).wait()
                _, idxv, r = steps[k]
                pltpu.sync_copy(bufs[k % nsb],
                                acc.at[idxv.at[r // NB, r % NB]], add=True)
                if k + nsb < L:
                    pend[k + nsb] = _issue(k + nsb)
            plsc.subcore_barrier()
            pltpu.sync_copy(acc.at[pl.ds(row0, ROWS_T)],
                            pooled.at[j, pl.ds(row0, ROWS_T)])
            return carry

        lax.fori_loop(0, CH // NC, chunk_body, 0)

    return scatter


@functools.cache
def _make_counts():
    @functools.partial(
        pl.kernel,
        out_type=jax.ShapeDtypeStruct((R_PAD, CW), F32),
        mesh=_sc_mesh(),
        compiler_params=pltpu.CompilerParams(use_tc_tiling_on_sc=False),
        scratch_types=[
            pltpu.VMEM_SHARED((R_PAD, CW), F32),
            pltpu.VMEM((2, NBP, EB), I32),
            pltpu.VMEM((2, NBP, EB), I32),
            pltpu.VMEM((EB, CW), F32),
        ],
    )
    def countk(sidx, oidx, zrows, ones, counts, acc, idx_sv, idx_ov, obuf):
        c = lax.axis_index("c")
        s = lax.axis_index("s")
        pltpu.sync_copy(sidx.at[pl.ds(s * 2, 2)], idx_sv)
        pltpu.sync_copy(oidx.at[pl.ds(s * 2, 2)], idx_ov)
        pltpu.sync_copy(ones, obuf)
        row0 = s * ROWS_T
        pltpu.sync_copy(zrows, acc.at[pl.ds(row0, ROWS_T)])
        plsc.subcore_barrier()

        def _accumulate():
            for idxv in (idx_sv, idx_ov):
                for r in range(2 * NB):
                    pltpu.sync_copy(obuf, acc.at[idxv.at[r // NB, r % NB]], add=True)

        pl.when(c == 0)(_accumulate)
        plsc.subcore_barrier()

        def _writeout():
            pltpu.sync_copy(acc.at[pl.ds(row0, ROWS_T)],
                            counts.at[pl.ds(row0, ROWS_T)])

        pl.when(c == 0)(_writeout)

    return countk


def _dot(a, b):
    return jnp.dot(a, b, preferred_element_type=F32)


@functools.cache
def _make_net1(din, dout, has_p, has_hi=False):
    def body(gs, pv, go, *args):
        if has_hi:
            gsh, goh, w1sh, w1oh, *args = args
        w1s, w1p, w1o, b1, w2s, b2s, *rest = args
        if has_p:
            w2p, b2p, w2o, b2o, ns_ref, np_ref, no_ref = rest
        else:
            w2o, b2o, ns_ref, no_ref = rest
        h = _dot(gs[...], w1s[...]) + _dot(pv[...], w1p[...]) \
            + _dot(go[...], w1o[...]) + b1[...]
        if has_hi:
            h = h + _dot(gsh[...], w1sh[...]) + _dot(goh[...], w1oh[...])
        h = jnp.maximum(h, 0.0)
        ns = jnp.maximum(_dot(h, w2s[...]) + b2s[...], 0.0)
        for j in range(CH):
            ns_ref[j] = jnp.concatenate(
                [ns[128 * a:128 * (a + 1), j * CW:(j + 1) * CW] for a in range(4)],
                axis=1)
        if has_p:
            np_ref[...] = jnp.maximum(_dot(h, w2p[...]) + b2p[...], 0.0)
        no = jnp.maximum(_dot(h, w2o[...]) + b2o[...], 0.0)
        for j in range(CH):
            no_ref[j] = jnp.concatenate(
                [no[128 * a:128 * (a + 1), j * CW:(j + 1) * CW] for a in range(4)],
                axis=1)

    full = lambda shape: pl.BlockSpec(shape, lambda i: (0,) * len(shape))
    in_specs = [
        pl.BlockSpec((BN, din), lambda i: (i, 0)),
        pl.BlockSpec((BN, D_PRED), lambda i: (i, 0)),
        pl.BlockSpec((BN, din), lambda i: (i, 0)),
    ]
    if has_hi:
        in_specs += [
            pl.BlockSpec((BN, D_HI), lambda i: (i, 0)),
            pl.BlockSpec((BN, D_HI), lambda i: (i, 0)),
            full((D_HI, HID)), full((D_HI, HID)),
        ]
    in_specs += [
        full((din, HID)), full((D_PRED, HID)), full((din, HID)), full((1, HID)),
        full((HID, HID)), full((1, HID)),
    ]
    out_shapes = [jax.ShapeDtypeStruct((CH, E4, 128), F32)]
    out_specs = [pl.BlockSpec((CH, BN // 4, 128), lambda i: (0, i, 0))]
    if has_p:
        in_specs += [full((HID, dout)), full((1, dout))]
        out_shapes.append(jax.ShapeDtypeStruct((E_PAD, dout), F32))
        out_specs.append(pl.BlockSpec((BN, dout), lambda i: (i, 0)))
    in_specs += [full((HID, HID)), full((1, HID))]
    out_shapes.append(jax.ShapeDtypeStruct((CH, E4, 128), F32))
    out_specs.append(pl.BlockSpec((CH, BN // 4, 128), lambda i: (0, i, 0)))

    return pl.pallas_call(
        body,
        grid=(E_PAD // BN,),
        in_specs=in_specs,
        out_specs=out_specs,
        out_shape=out_shapes,
    )


@functools.cache
def _make_net2(dout, unpermute):
    def _unpack(blk):
        return jnp.concatenate(
            [blk[:, 32 * a:32 * (a + 1)] for a in range(4)], axis=0)

    def body(pc, cnt, w3, b3, w4, b4, *rest):
        if unpermute:
            pt, out_ref = rest
        else:
            (out_ref,) = rest
        x = jnp.concatenate([_unpack(pc[j]) for j in range(CH)], axis=1)
        cnt0 = jnp.concatenate(
            [cnt[:, 32 * a:32 * a + 1] for a in range(4)], axis=0)
        scale = 1.0 / jnp.maximum(cnt0, 1.0)
        h = jnp.maximum(_dot(x * scale, w3[...]) + b3[...], 0.0)
        y = jnp.maximum(_dot(h, w4[...]) + b4[...], 0.0)
        if unpermute:
            y = _dot(pt[...], y)
        out_ref[...] = y

    full = lambda shape: pl.BlockSpec(shape, lambda i: (0,) * len(shape))
    in_specs = [
        pl.BlockSpec((CH, BN // 4, 128), lambda i: (0, i, 0)),
        pl.BlockSpec((BN // 4, 128), lambda i: (i, 0)),
        full((HID, HID)), full((1, HID)),
        full((HID, dout)), full((1, dout)),
    ]
    if unpermute:
        in_specs.append(full((BN, BN)))
    return pl.pallas_call(
        body,
        grid=(R_PAD // BN,),
        in_specs=in_specs,
        out_specs=pl.BlockSpec((BN, dout), lambda i: (i, 0)),
        out_shape=jax.ShapeDtypeStruct((R_PAD, dout), F32),
    )


def _layer_weights(p, din_raw, dout):
    (w1, b1), (w2, b2) = p["net1"]
    w1s = w1[:din_raw]
    w1p = w1[din_raw:din_raw + D_PRED]
    w1o = w1[din_raw + D_PRED:]
    hi = ()
    if din_raw > D_PRED:
        pad = ((0, D_HI - (din_raw - D_PRED)), (0, 0))
        hi = (jnp.pad(w1s[D_PRED:], pad), jnp.pad(w1o[D_PRED:], pad))
        w1s, w1o = w1s[:D_PRED], w1o[:D_PRED]
    w2s, b2s = w2[:, :HID], b2[:HID]
    w2p, b2p = w2[:, HID:HID + dout], b2[HID:HID + dout]
    w2o, b2o = w2[:, HID + dout:], b2[HID + dout:]
    (w3, b3), (w4, b4) = p["net2"]
    return (w1s, w1p, w1o, b1.reshape(1, -1),
            w2s, b2s.reshape(1, -1), w2p, b2p.reshape(1, -1),
            w2o, b2o.reshape(1, -1),
            w3, b3.reshape(1, -1), w4, b4.reshape(1, -1), hi)


def _ilv(x):
    return (x // 512) * 512 + 128 * (x % 4) + (x % 512) // 4


def kernel(obj_vecs, pred_vecs, triples, params):
    b, obj = obj_vecs.shape[0], obj_vecs.shape[1]
    n = b * obj
    ov = obj_vecs.reshape(n, -1)
    pv = pred_vecs.reshape(n, -1)
    count = jnp.arange(0, n, obj, dtype=triples.dtype)[:, None, None]
    tr = (triples + count).reshape(n, 3)
    sflat = jnp.full((E_PAD,), DEAD, I32).at[:n].set(tr[:, 0])
    oflat = jnp.full((E_PAD,), DEAD, I32).at[:n].set(tr[:, 2])
    def _rows(x):
        x = x.reshape(NW, NB, EB)
        return jnp.pad(x, ((0, 0), (0, NBP - NB), (0, 0)))

    sidx_g0 = _rows(sflat)
    oidx_g0 = _rows(oflat)
    sidx_g = _rows(_ilv(sflat))
    oidx_g = _rows(_ilv(oflat))
    psc = _ilv(jnp.arange(E_PAD, dtype=I32))
    sidx_sc = _rows(sflat[psc])
    oidx_sc = _rows(oflat[psc])
    zrows = jnp.zeros((ROWS_T, CW), F32)
    ones = jnp.ones((EB, CW), F32)
    pt = jax.nn.one_hot(_ilv(jnp.arange(BN)), BN, dtype=F32)

    counts = _make_counts()(sidx_sc, oidx_sc, zrows, ones).reshape(R4, 128)

    table = jnp.pad(ov[:, :D_PRED], ((0, R_PAD - n), (0, 0)))
    table_hi = jnp.pad(ov[:, D_PRED:], ((0, R_PAD - n), (0, D_HI - (D_OBJ0 - D_PRED))))
    pvp = jnp.pad(pv, ((0, E_PAD - n), (0, 0)))
    douts = [D_OUT] * 5 + [D_OBJ0]
    dins_raw = [D_OBJ0] + [D_OUT] * 5

    for li, (p, dout) in enumerate(zip(params, douts)):
        din_raw = dins_raw[li]
        has_p = li < 5
        last = li == 5
        has_hi = li == 0
        (w1s, w1p, w1o, b1, w2s, b2s, w2p, b2p, w2o, b2o,
         w3, b3, w4, b4, hi) = _layer_weights(p, din_raw, dout)
        gsi, goi = (sidx_g0, oidx_g0) if li == 0 else (sidx_g, oidx_g)
        hi_args = ()
        if has_hi:
            gs, go, gsh, goh = _make_gather(D_PRED, D_HI)(table, table_hi,
                                                          gsi, goi)
            hi_args = (gsh, goh) + hi
        else:
            gs, go = _make_gather(D_PRED)(table, gsi, goi)
        if has_p:
            ns, np_, no = _make_net1(D_PRED, dout, True, has_hi)(
                gs, pvp, go, *hi_args,
                w1s, w1p, w1o, b1, w2s, b2s, w2p, b2p, w2o, b2o)
            pvp = np_
        else:
            ns, no = _make_net1(D_PRED, dout, False, has_hi)(
                gs, pvp, go, *hi_args, w1s, w1p, w1o, b1, w2s, b2s, w2o, b2o)
        pooled = _make_scatter()(ns.reshape(CH, E_PAD, CW),
                                 no.reshape(CH, E_PAD, CW),
                                 sidx_sc, oidx_sc, zrows)
        net2_args = (pooled.reshape(CH, R4, 128), counts, w3, b3, w4, b4)
        if last:
            net2_args += (pt,)
        table = _make_net2(dout, last)(*net2_args)

    return table[:n].reshape(b, obj, -1)


# speedup vs baseline: 1.0393x; 1.0393x over previous
import functools

import jax
import jax.numpy as jnp
from jax import lax
from jax.experimental import pallas as pl
from jax.experimental.pallas import tpu as pltpu
from jax.experimental.pallas import tpu_sc as plsc

F32 = jnp.float32
I32 = jnp.int32

D_OBJ0 = 132
D_HI = 16
D_PRED = 128
D_OUT = 128
HID = 512

NC, NS = 2, 16
NW = NC * NS
EB = 128
NB = 13
E_W = EB * NB
E_PAD = NW * E_W
R_PAD = 50176
ROWS_T = R_PAD // NS
DEAD = 50100
CH = 16
CW = 32
BN = 512

NBP = 16

E4 = E_PAD // 4
R4 = R_PAD // 4


def _sc_mesh():
    return plsc.VectorSubcoreMesh(core_axis_name="c", subcore_axis_name="s",
                                  num_cores=NC, num_subcores=NS)


@functools.cache
def _make_gather(d, dh=None):
    nbuf = 6
    outs = [jax.ShapeDtypeStruct((E_PAD, d), F32)] * 2
    scratch = [
        pltpu.VMEM((1, NBP, EB), I32),
        pltpu.VMEM((1, NBP, EB), I32),
    ] + [pltpu.VMEM((EB, d), F32)] * nbuf + [pltpu.SemaphoreType.DMA] * nbuf
    if dh is not None:
        outs += [jax.ShapeDtypeStruct((E_PAD, dh), F32)] * 2
        scratch += [pltpu.VMEM((EB, dh), F32)] * nbuf \
            + [pltpu.SemaphoreType.DMA] * nbuf

    @functools.partial(
        pl.kernel,
        out_type=tuple(outs),
        mesh=_sc_mesh(),
        compiler_params=pltpu.CompilerParams(use_tc_tiling_on_sc=False),
        scratch_types=scratch,
    )
    def gather(*args):
        if dh is not None:
            table, table_h, sidx, oidx, gs, go, gsh, goh, idx_sv, idx_ov, *bs = args
            bufs_h = bs[2 * nbuf:3 * nbuf]
            sems_h = bs[3 * nbuf:]
        else:
            table, sidx, oidx, gs, go, idx_sv, idx_ov, *bs = args
        bufs = bs[:nbuf]
        sems_in = bs[nbuf:2 * nbuf]
        c = lax.axis_index("c")
        s = lax.axis_index("s")
        wid = c * NS + s
        pltpu.sync_copy(sidx.at[pl.ds(wid, 1)], idx_sv)
        pltpu.sync_copy(oidx.at[pl.ds(wid, 1)], idx_ov)
        base = wid * E_W

        def _pipeline(tbl, steps, rbufs, rsems):
            L = len(steps)

            def _issue(k):
                idxv, _, b = steps[k]
                return pltpu.async_copy(tbl.at[idxv.at[0, b]],
                                        rbufs[k % nbuf], rsems[k % nbuf])

            pend = {k: _issue(k) for k in range(min(nbuf, L))}
            for k in range(L):
                pend.pop(k).wait()
                _, out, b = steps[k]
                pltpu.sync_copy(rbufs[k % nbuf],
                                out.at[pl.ds(base + b * EB, EB)])
                if k + nbuf < L:
                    pend[k + nbuf] = _issue(k + nbuf)

        _pipeline(table,
                  [(idx_sv, gs, b) for b in range(NB)]
                  + [(idx_ov, go, b) for b in range(NB)], bufs, sems_in)
        if dh is not None:
            _pipeline(table_h,
                      [(idx_sv, gsh, b) for b in range(NB)]
                      + [(idx_ov, goh, b) for b in range(NB)], bufs_h, sems_h)

    return gather


@functools.cache
def _make_scatter():
    nsb = 4

    @functools.partial(
        pl.kernel,
        out_type=jax.ShapeDtypeStruct((CH, R_PAD, CW), F32),
        mesh=_sc_mesh(),
        compiler_params=pltpu.CompilerParams(use_tc_tiling_on_sc=False),
        scratch_types=[
            pltpu.VMEM_SHARED((R_PAD, CW), F32),
            pltpu.VMEM((2, NBP, EB), I32),
            pltpu.VMEM((2, NBP, EB), I32),
        ] + [pltpu.VMEM((EB, CW), F32)] * nsb
          + [pltpu.SemaphoreType.DMA] * nsb,
    )
    def scatter(src_s, src_o, sidx, oidx, zrows, pooled,
                acc, idx_sv, idx_ov, *bufsems):
        bufs = bufsems[:nsb]
        sems_st = bufsems[nsb:]
        c = lax.axis_index("c")
        s = lax.axis_index("s")
        pltpu.sync_copy(sidx.at[pl.ds(s * 2, 2)], idx_sv)
        pltpu.sync_copy(oidx.at[pl.ds(s * 2, 2)], idx_ov)
        row0 = s * ROWS_T
        ebase = s * (2 * NB * EB)
        steps = [(src_s, idx_sv, r) for r in range(2 * NB)] + \
                [(src_o, idx_ov, r) for r in range(2 * NB)]
        L = len(steps)

        def chunk_body(jj, carry):
            j = c * (CH // NC) + jj
            pltpu.sync_copy(zrows, acc.at[pl.ds(row0, ROWS_T)])
            plsc.subcore_barrier()

            def _issue(k):
                src, _, r = steps[k]
                return pltpu.async_copy(src.at[j, pl.ds(ebase + r * EB, EB)],
                                        bufs[k % nsb], sems_st[k % nsb])

            pend = {k: _issue(k) for k in range(min(nsb, L))}
            for k in range(L):
                pend.pop(k).wait()
                _, idxv, r = steps[k]
                pltpu.sync_copy(bufs[k % nsb],
                                acc.at[idxv.at[r // NB, r % NB]], add=True)
                if k + nsb < L:
                    pend[k + nsb] = _issue(k + nsb)
            plsc.subcore_barrier()
            pltpu.sync_copy(acc.at[pl.ds(row0, ROWS_T)],
                            pooled.at[j, pl.ds(row0, ROWS_T)])
            return carry

        lax.fori_loop(0, CH // NC, chunk_body, 0)

    return scatter


@functools.cache
def _make_counts():
    @functools.partial(
        pl.kernel,
        out_type=jax.ShapeDtypeStruct((R_PAD, CW), F32),
        mesh=_sc_mesh(),
        compiler_params=pltpu.CompilerParams(use_tc_tiling_on_sc=False),
        scratch_types=[
            pltpu.VMEM_SHARED((R_PAD, CW), F32),
            pltpu.VMEM((2, NBP, EB), I32),
            pltpu.VMEM((2, NBP, EB), I32),
            pltpu.VMEM((EB, CW), F32),
        ],
    )
    def countk(sidx, oidx, zrows, ones, counts, acc, idx_sv, idx_ov, obuf):
        c = lax.axis_index("c")
        s = lax.axis_index("s")
        pltpu.sync_copy(sidx.at[pl.ds(s * 2, 2)], idx_sv)
        pltpu.sync_copy(oidx.at[pl.ds(s * 2, 2)], idx_ov)
        pltpu.sync_copy(ones, obuf)
        row0 = s * ROWS_T
        pltpu.sync_copy(zrows, acc.at[pl.ds(row0, ROWS_T)])
        plsc.subcore_barrier()

        def _accumulate():
            for idxv in (idx_sv, idx_ov):
                for r in range(2 * NB):
                    pltpu.sync_copy(obuf, acc.at[idxv.at[r // NB, r % NB]], add=True)

        pl.when(c == 0)(_accumulate)
        plsc.subcore_barrier()

        def _writeout():
            pltpu.sync_copy(acc.at[pl.ds(row0, ROWS_T)],
                            counts.at[pl.ds(row0, ROWS_T)])

        pl.when(c == 0)(_writeout)

    return countk


def _dot(a, b):
    return jnp.dot(a, b, preferred_element_type=F32)


@functools.cache
def _make_net1(din, dout, has_p, has_hi=False):
    def body(gs, pv, go, *args):
        if has_hi:
            gsh, goh, w1sh, w1oh, *args = args
        w1s, w1p, w1o, b1, w2s, b2s, *rest = args
        if has_p:
            w2p, b2p, w2o, b2o, ns_ref, np_ref, no_ref = rest
        else:
            w2o, b2o, ns_ref, no_ref = rest
        h = _dot(gs[...], w1s[...]) + _dot(pv[...], w1p[...]) \
            + _dot(go[...], w1o[...]) + b1[...]
        if has_hi:
            h = h + _dot(gsh[...], w1sh[...]) + _dot(goh[...], w1oh[...])
        h = jnp.maximum(h, 0.0)
        ns = jnp.maximum(_dot(h, w2s[...]) + b2s[...], 0.0)
        for j in range(CH):
            ns_ref[j] = jnp.concatenate(
                [ns[128 * a:128 * (a + 1), j * CW:(j + 1) * CW] for a in range(4)],
                axis=1)
        if has_p:
            np_ref[...] = jnp.maximum(_dot(h, w2p[...]) + b2p[...], 0.0)
        no = jnp.maximum(_dot(h, w2o[...]) + b2o[...], 0.0)
        for j in range(CH):
            no_ref[j] = jnp.concatenate(
                [no[128 * a:128 * (a + 1), j * CW:(j + 1) * CW] for a in range(4)],
                axis=1)

    full = lambda shape: pl.BlockSpec(shape, lambda i: (0,) * len(shape))
    in_specs = [
        pl.BlockSpec((BN, din), lambda i: (i, 0)),
        pl.BlockSpec((BN, D_PRED), lambda i: (i, 0)),
        pl.BlockSpec((BN, din), lambda i: (i, 0)),
    ]
    if has_hi:
        in_specs += [
            pl.BlockSpec((BN, D_HI), lambda i: (i, 0)),
            pl.BlockSpec((BN, D_HI), lambda i: (i, 0)),
            full((D_HI, HID)), full((D_HI, HID)),
        ]
    in_specs += [
        full((din, HID)), full((D_PRED, HID)), full((din, HID)), full((1, HID)),
        full((HID, HID)), full((1, HID)),
    ]
    out_shapes = [jax.ShapeDtypeStruct((CH, E4, 128), F32)]
    out_specs = [pl.BlockSpec((CH, BN // 4, 128), lambda i: (0, i, 0))]
    if has_p:
        in_specs += [full((HID, dout)), full((1, dout))]
        out_shapes.append(jax.ShapeDtypeStruct((E_PAD, dout), F32))
        out_specs.append(pl.BlockSpec((BN, dout), lambda i: (i, 0)))
    in_specs += [full((HID, HID)), full((1, HID))]
    out_shapes.append(jax.ShapeDtypeStruct((CH, E4, 128), F32))
    out_specs.append(pl.BlockSpec((CH, BN // 4, 128), lambda i: (0, i, 0)))

    return pl.pallas_call(
        body,
        grid=(E_PAD // BN,),
        in_specs=in_specs,
        out_specs=out_specs,
        out_shape=out_shapes,
    )


@functools.cache
def _make_net2(dout, unpermute):
    def _unpack(blk):
        return jnp.concatenate(
            [blk[:, 32 * a:32 * (a + 1)] for a in range(4)], axis=0)

    def body(pc, cnt, w3, b3, w4, b4, *rest):
        if unpermute:
            pt, out_ref = rest
        else:
            (out_ref,) = rest
        x = jnp.concatenate([_unpack(pc[j]) for j in range(CH)], axis=1)
        scale = 1.0 / jnp.maximum(_unpack(cnt[...])[:, 0:1], 1.0)
        h = jnp.maximum(_dot(x * scale, w3[...]) + b3[...], 0.0)
        y = jnp.maximum(_dot(h, w4[...]) + b4[...], 0.0)
        if unpermute:
            y = _dot(pt[...], y)
        out_ref[...] = y

    full = lambda shape: pl.BlockSpec(shape, lambda i: (0,) * len(shape))
    in_specs = [
        pl.BlockSpec((CH, BN // 4, 128), lambda i: (0, i, 0)),
        pl.BlockSpec((BN // 4, 128), lambda i: (i, 0)),
        full((HID, HID)), full((1, HID)),
        full((HID, dout)), full((1, dout)),
    ]
    if unpermute:
        in_specs.append(full((BN, BN)))
    return pl.pallas_call(
        body,
        grid=(R_PAD // BN,),
        in_specs=in_specs,
        out_specs=pl.BlockSpec((BN, dout), lambda i: (i, 0)),
        out_shape=jax.ShapeDtypeStruct((R_PAD, dout), F32),
    )


def _layer_weights(p, din_raw, dout):
    (w1, b1), (w2, b2) = p["net1"]
    w1s = w1[:din_raw]
    w1p = w1[din_raw:din_raw + D_PRED]
    w1o = w1[din_raw + D_PRED:]
    hi = ()
    if din_raw > D_PRED:
        pad = ((0, D_HI - (din_raw - D_PRED)), (0, 0))
        hi = (jnp.pad(w1s[D_PRED:], pad), jnp.pad(w1o[D_PRED:], pad))
        w1s, w1o = w1s[:D_PRED], w1o[:D_PRED]
    w2s, b2s = w2[:, :HID], b2[:HID]
    w2p, b2p = w2[:, HID:HID + dout], b2[HID:HID + dout]
    w2o, b2o = w2[:, HID + dout:], b2[HID + dout:]
    (w3, b3), (w4, b4) = p["net2"]
    return (w1s, w1p, w1o, b1.reshape(1, -1),
            w2s, b2s.reshape(1, -1), w2p, b2p.reshape(1, -1),
            w2o, b2o.reshape(1, -1),
            w3, b3.reshape(1, -1), w4, b4.reshape(1, -1), hi)


def _ilv(x):
    return (x // 512) * 512 + 128 * (x % 4) + (x % 512) // 4


def kernel(obj_vecs, pred_vecs, triples, params):
    b, obj = obj_vecs.shape[0], obj_vecs.shape[1]
    n = b * obj
    ov = obj_vecs.reshape(n, -1)
    pv = pred_vecs.reshape(n, -1)
    count = jnp.arange(0, n, obj, dtype=triples.dtype)[:, None, None]
    tr = (triples + count).reshape(n, 3)
    sflat = jnp.full((E_PAD,), DEAD, I32).at[:n].set(tr[:, 0])
    oflat = jnp.full((E_PAD,), DEAD, I32).at[:n].set(tr[:, 2])
    def _rows(x):
        x = x.reshape(NW, NB, EB)
        return jnp.pad(x, ((0, 0), (0, NBP - NB), (0, 0)))

    sidx_g0 = _rows(sflat)
    oidx_g0 = _rows(oflat)
    sidx_g = _rows(_ilv(sflat))
    oidx_g = _rows(_ilv(oflat))
    psc = _ilv(jnp.arange(E_PAD, dtype=I32))
    sidx_sc = _rows(sflat[psc])
    oidx_sc = _rows(oflat[psc])
    zrows = jnp.zeros((ROWS_T, CW), F32)
    ones = jnp.ones((EB, CW), F32)
    pt = jax.nn.one_hot(_ilv(jnp.arange(BN)), BN, dtype=F32)

    counts = _make_counts()(sidx_sc, oidx_sc, zrows, ones).reshape(R4, 128)

    table = jnp.pad(ov[:, :D_PRED], ((0, R_PAD - n), (0, 0)))
    table_hi = jnp.pad(ov[:, D_PRED:], ((0, R_PAD - n), (0, D_HI - (D_OBJ0 - D_PRED))))
    pvp = jnp.pad(pv, ((0, E_PAD - n), (0, 0)))
    douts = [D_OUT] * 5 + [D_OBJ0]
    dins_raw = [D_OBJ0] + [D_OUT] * 5

    for li, (p, dout) in enumerate(zip(params, douts)):
        din_raw = dins_raw[li]
        has_p = li < 5
        last = li == 5
        has_hi = li == 0
        (w1s, w1p, w1o, b1, w2s, b2s, w2p, b2p, w2o, b2o,
         w3, b3, w4, b4, hi) = _layer_weights(p, din_raw, dout)
        gsi, goi = (sidx_g0, oidx_g0) if li == 0 else (sidx_g, oidx_g)
        gs, go = _make_gather(D_PRED)(table, gsi, goi)
        hi_args = ()
        if has_hi:
            gsh, goh = _make_gather(D_HI)(table_hi, gsi, goi)
            hi_args = (gsh, goh) + hi
        if has_p:
            ns, np_, no = _make_net1(D_PRED, dout, True, has_hi)(
                gs, pvp, go, *hi_args,
                w1s, w1p, w1o, b1, w2s, b2s, w2p, b2p, w2o, b2o)
            pvp = np_
        else:
            ns, no = _make_net1(D_PRED, dout, False, has_hi)(
                gs, pvp, go, *hi_args, w1s, w1p, w1o, b1, w2s, b2s, w2o, b2o)
        pooled = _make_scatter()(ns.reshape(CH, E_PAD, CW),
                                 no.reshape(CH, E_PAD, CW),
                                 sidx_sc, oidx_sc, zrows)
        net2_args = (pooled.reshape(CH, R4, 128), counts, w3, b3, w4, b4)
        if last:
            net2_args += (pt,)
        table = _make_net2(dout, last)(*net2_args)

    return table[:n].reshape(b, obj, -1)
